# baseline (speedup 1.0000x reference)
import jax
import jax.numpy as jnp
from jax.experimental import pallas as pl
from jax.experimental.pallas import tpu as pltpu
from jax.experimental.pallas import tpu_sc as plsc

N_TOK = 65536
N_EMB = 1024
DIM = 64
BLK = 1024
NCHUNK = 2
CHT = N_TOK // NCHUNK

SC_NC = 2
SC_NS = 16
SC_NW = SC_NC * SC_NS
SC_CH = 128


def _tc_body(xt_ref, w_ref, idx_ref, loss_ref, acc_ref):
    i = pl.program_id(0)
    xt = xt_ref[...]
    w = w_ref[...]
    c = jax.lax.dot_general(w, xt, (((1,), (0,)), ((), ())),
                            preferred_element_type=jnp.float32)
    a = jnp.sum(xt * xt, axis=0, keepdims=True)
    b = jnp.sum(w * w, axis=1, keepdims=True)
    dist = (b + a) - 2.0 * c
    jidx = jax.lax.broadcasted_iota(
        jnp.int32, (N_EMB, 128), 0).astype(jnp.float32)
    idx_rows = []
    tot = jnp.float32(0.0)
    for g8 in range(BLK // 128):
        sl = dist[:, g8 * 128:(g8 + 1) * 128]
        mg = jnp.min(sl, axis=0, keepdims=True)
        ig = jnp.min(jnp.where(sl == mg, jidx, float(N_EMB)),
                     axis=0, keepdims=True)
        idx_rows.append(ig)
        tot = tot + jnp.sum(mg)
    idx_ref[...] = jnp.concatenate(idx_rows, axis=0).astype(jnp.int32)

    @pl.when(i == 0)
    def _():
        acc_ref[0] = 0.0

    acc_ref[0] += tot

    @pl.when(i == pl.num_programs(0) - 1)
    def _():
        loss_ref[...] = jnp.full((1, 1), acc_ref[0], dtype=jnp.float32)


def _tc_argmin_loss(xt, weight, k):
    g = CHT // BLK
    return pl.pallas_call(
        _tc_body,
        grid=(g,),
        in_specs=[
            pl.BlockSpec((DIM, BLK), lambda i, k=k: (0, i + k * g)),
            pl.BlockSpec((N_EMB, DIM), lambda i: (0, 0)),
        ],
        out_specs=[
            pl.BlockSpec((BLK // 128, 128), lambda i: (i, 0)),
            pl.BlockSpec((1, 1), lambda i: (0, 0)),
        ],
        out_shape=[
            jax.ShapeDtypeStruct((CHT // 128, 128), jnp.int32),
            jax.ShapeDtypeStruct((1, 1), jnp.float32),
        ],
        scratch_shapes=[pltpu.SMEM((1,), jnp.float32)],
    )(xt, weight)


def _sc_gather(w_pad, idx):
    n = idx.shape[0]
    rows_per_w = n // SC_NW
    n_ch = rows_per_w // SC_CH
    mesh = plsc.VectorSubcoreMesh(core_axis_name="c", subcore_axis_name="s")

    nb = 4

    @pl.kernel(out_type=jax.ShapeDtypeStruct((n, 128), jnp.float32),
               mesh=mesh,
               scratch_types=(
                   [pltpu.VMEM((rows_per_w,), jnp.int32)]
                   + [pltpu.VMEM((SC_CH, 128), jnp.float32)] * nb
                   + [pltpu.SemaphoreType.DMA] * (2 * nb)
               ))
    def k(w_hbm, i_hbm, o_hbm, idx_all, *bufs_sems):
        bufs = bufs_sems[:nb]
        gsems = bufs_sems[nb:2 * nb]
        wsems = bufs_sems[2 * nb:]
        wid = jax.lax.axis_index("s") * SC_NC + jax.lax.axis_index("c")
        base = wid * rows_per_w
        pltpu.sync_copy(i_hbm.at[pl.ds(base, rows_per_w)], idx_all)
        gathers = [None] * n_ch
        writes = [None] * n_ch
        for c in range(n_ch):
            s = c % nb
            if c >= nb:
                writes[c - nb].wait()
            gathers[c] = pltpu.async_copy(
                w_hbm.at[idx_all.at[pl.ds(c * SC_CH, SC_CH)]], bufs[s],
                gsems[s])
            if c >= 1:
                d = c - 1
                gathers[d].wait()
                writes[d] = pltpu.async_copy(
                    bufs[d % nb],
                    o_hbm.at[pl.ds(base + d * SC_CH, SC_CH)],
                    wsems[d % nb])
        d = n_ch - 1
        gathers[d].wait()
        writes[d] = pltpu.async_copy(
            bufs[d % nb], o_hbm.at[pl.ds(base + d * SC_CH, SC_CH)],
            wsems[d % nb])
        for d in range(max(0, n_ch - nb), n_ch):
            writes[d].wait()

    return k(w_pad, idx)


SL_BLK = 4096


def _packT_body(_, q_ref, o_ref):
    o_ref[...] = q_ref[:, :DIM].T


def _packT_first(q_raw):
    g = CHT // SL_BLK
    return pl.pallas_call(
        lambda q, o: _packT_body(None, q, o),
        grid=(g,),
        in_specs=[pl.BlockSpec((SL_BLK, 128), lambda i: (i, 0))],
        out_specs=pl.BlockSpec((DIM, SL_BLK), lambda i: (0, i)),
        out_shape=jax.ShapeDtypeStruct((DIM, N_TOK), jnp.float32),
    )(q_raw)


def _packT_into(buf, q_raw, k):
    g = CHT // SL_BLK
    return pl.pallas_call(
        _packT_body,
        grid=(g,),
        in_specs=[
            pl.BlockSpec(memory_space=pl.ANY),
            pl.BlockSpec((SL_BLK, 128), lambda i: (i, 0)),
        ],
        out_specs=pl.BlockSpec((DIM, SL_BLK), lambda i, k=k, g=g: (0, i + k * g)),
        out_shape=jax.ShapeDtypeStruct((DIM, N_TOK), jnp.float32),
        input_output_aliases={0: 0},
    )(buf, q_raw)


def kernel(inputs, weight):
    w_pad = jnp.concatenate(
        [weight, jnp.zeros((N_EMB, 128 - DIM), jnp.float32)], axis=1)
    xt = inputs.T
    idx_parts, loss_parts, q_raws = [], [], []
    for k in range(NCHUNK):
        idx2d, lsum = _tc_argmin_loss(xt, weight, k)
        idx = idx2d.reshape(CHT)
        q_raws.append(_sc_gather(w_pad, idx))
        idx_parts.append(idx)
        loss_parts.append(lsum[0, 0])
    qt = _packT_first(q_raws[0])
    for k in range(1, NCHUNK):
        qt = _packT_into(qt, q_raws[k], k)
    quantized = qt.T
    loss = sum(loss_parts) * (1.25 / (N_TOK * DIM))
    indices = jnp.concatenate(idx_parts, axis=0)
    return loss, quantized, indices


# speedup vs baseline: 1.0882x; 1.0882x over previous
import jax
import jax.numpy as jnp
from jax.experimental import pallas as pl
from jax.experimental.pallas import tpu as pltpu
from jax.experimental.pallas import tpu_sc as plsc

N_TOK = 65536
N_EMB = 1024
DIM = 64
BLK = 2048
NCHUNK = 2
CHT = N_TOK // NCHUNK

SC_NC = 2
SC_NS = 16
SC_NW = SC_NC * SC_NS
SC_CH = 128


def _tc_body(xt_ref, w_ref, idx_ref, loss_ref, acc_ref):
    i = pl.program_id(0)
    xt = xt_ref[...]
    w = w_ref[...]
    c = jax.lax.dot_general(w, xt, (((1,), (0,)), ((), ())),
                            preferred_element_type=jnp.float32)
    a = jnp.sum(xt * xt, axis=0, keepdims=True)
    b = jnp.sum(w * w, axis=1, keepdims=True)
    dist = (b + a) - 2.0 * c
    m = jnp.min(dist, axis=0, keepdims=True)
    jidx = jax.lax.broadcasted_iota(
        jnp.int32, dist.shape, 0).astype(jnp.float32)
    idxf = jnp.min(jnp.where(dist == m, jidx, float(N_EMB)), axis=0)
    idx_ref[...] = idxf.astype(jnp.int32).reshape(1, 1, BLK)

    @pl.when(i == 0)
    def _():
        acc_ref[0] = 0.0

    acc_ref[0] += jnp.sum(m)

    @pl.when(i == pl.num_programs(0) - 1)
    def _():
        loss_ref[...] = jnp.full((1, 1), acc_ref[0], dtype=jnp.float32)


def _tc_argmin_loss(xt, weight, k):
    g = CHT // BLK
    return pl.pallas_call(
        _tc_body,
        grid=(g,),
        in_specs=[
            pl.BlockSpec((DIM, BLK), lambda i, k=k: (0, i + k * g)),
            pl.BlockSpec((N_EMB, DIM), lambda i: (0, 0)),
        ],
        out_specs=[
            pl.BlockSpec((1, 1, BLK), lambda i: (i, 0, 0)),
            pl.BlockSpec((1, 1), lambda i: (0, 0)),
        ],
        out_shape=[
            jax.ShapeDtypeStruct((g, 1, BLK), jnp.int32),
            jax.ShapeDtypeStruct((1, 1), jnp.float32),
        ],
        scratch_shapes=[pltpu.SMEM((1,), jnp.float32)],
    )(xt, weight)


def _sc_gather(w_pad, idx):
    n = idx.shape[0]
    rows_per_w = n // SC_NW
    n_ch = rows_per_w // SC_CH
    mesh = plsc.VectorSubcoreMesh(core_axis_name="c", subcore_axis_name="s")

    nb = 4

    @pl.kernel(out_type=jax.ShapeDtypeStruct((n, 128), jnp.float32),
               mesh=mesh,
               scratch_types=(
                   [pltpu.VMEM((rows_per_w,), jnp.int32)]
                   + [pltpu.VMEM((SC_CH, 128), jnp.float32)] * nb
                   + [pltpu.SemaphoreType.DMA] * (2 * nb)
               ))
    def k(w_hbm, i_hbm, o_hbm, idx_all, *bufs_sems):
        bufs = bufs_sems[:nb]
        gsems = bufs_sems[nb:2 * nb]
        wsems = bufs_sems[2 * nb:]
        wid = jax.lax.axis_index("s") * SC_NC + jax.lax.axis_index("c")
        base = wid * rows_per_w
        pltpu.sync_copy(i_hbm.at[pl.ds(base, rows_per_w)], idx_all)
        gathers = [None] * n_ch
        writes = [None] * n_ch
        for c in range(n_ch):
            s = c % nb
            if c >= nb:
                writes[c - nb].wait()
            gathers[c] = pltpu.async_copy(
                w_hbm.at[idx_all.at[pl.ds(c * SC_CH, SC_CH)]], bufs[s],
                gsems[s])
            if c >= 1:
                d = c - 1
                gathers[d].wait()
                writes[d] = pltpu.async_copy(
                    bufs[d % nb],
                    o_hbm.at[pl.ds(base + d * SC_CH, SC_CH)],
                    wsems[d % nb])
        d = n_ch - 1
        gathers[d].wait()
        writes[d] = pltpu.async_copy(
            bufs[d % nb], o_hbm.at[pl.ds(base + d * SC_CH, SC_CH)],
            wsems[d % nb])
        for d in range(max(0, n_ch - nb), n_ch):
            writes[d].wait()

    return k(w_pad, idx)


SL_BLK = 8192


def _packT_body(_, q_ref, o_ref):
    o_ref[...] = q_ref[:, :DIM].T


def _packT_first(q_raw):
    g = CHT // SL_BLK
    return pl.pallas_call(
        lambda q, o: _packT_body(None, q, o),
        grid=(g,),
        in_specs=[pl.BlockSpec((SL_BLK, 128), lambda i: (i, 0))],
        out_specs=pl.BlockSpec((DIM, SL_BLK), lambda i: (0, i)),
        out_shape=jax.ShapeDtypeStruct((DIM, N_TOK), jnp.float32),
    )(q_raw)


def _packT_into(buf, q_raw, k):
    g = CHT // SL_BLK
    return pl.pallas_call(
        _packT_body,
        grid=(g,),
        in_specs=[
            pl.BlockSpec(memory_space=pl.ANY),
            pl.BlockSpec((SL_BLK, 128), lambda i: (i, 0)),
        ],
        out_specs=pl.BlockSpec((DIM, SL_BLK), lambda i, k=k, g=g: (0, i + k * g)),
        out_shape=jax.ShapeDtypeStruct((DIM, N_TOK), jnp.float32),
        input_output_aliases={0: 0},
    )(buf, q_raw)


def kernel(inputs, weight):
    w_pad = jnp.concatenate(
        [weight, jnp.zeros((N_EMB, 128 - DIM), jnp.float32)], axis=1)
    xt = inputs.T
    idx_parts, loss_parts, q_raws = [], [], []
    for k in range(NCHUNK):
        idx2d, lsum = _tc_argmin_loss(xt, weight, k)
        idx = idx2d.reshape(CHT)
        q_raws.append(_sc_gather(w_pad, idx))
        idx_parts.append(idx)
        loss_parts.append(lsum[0, 0])
    qt = _packT_first(q_raws[0])
    for k in range(1, NCHUNK):
        qt = _packT_into(qt, q_raws[k], k)
    quantized = qt.T
    loss = sum(loss_parts) * (1.25 / (N_TOK * DIM))
    indices = jnp.concatenate(idx_parts, axis=0)
    return loss, quantized, indices


# speedup vs baseline: 1.1287x; 1.0372x over previous
import jax
import jax.numpy as jnp
from jax.experimental import pallas as pl
from jax.experimental.pallas import tpu as pltpu
from jax.experimental.pallas import tpu_sc as plsc

N_TOK = 65536
N_EMB = 1024
DIM = 64
BLK = 4096
NCHUNK = 2
CHT = N_TOK // NCHUNK

SC_NC = 2
SC_NS = 16
SC_NW = SC_NC * SC_NS
SC_CH = 128


def _tc_body(xt_ref, w_ref, idx_ref, loss_ref, acc_ref):
    i = pl.program_id(0)
    xt = xt_ref[...]
    w = w_ref[...]
    c = jax.lax.dot_general(w, xt, (((1,), (0,)), ((), ())),
                            preferred_element_type=jnp.float32)
    a = jnp.sum(xt * xt, axis=0, keepdims=True)
    b = jnp.sum(w * w, axis=1, keepdims=True)
    dist = (b + a) - 2.0 * c
    m = jnp.min(dist, axis=0, keepdims=True)
    jidx = jax.lax.broadcasted_iota(
        jnp.int32, dist.shape, 0).astype(jnp.float32)
    idxf = jnp.min(jnp.where(dist == m, jidx, float(N_EMB)), axis=0)
    idx_ref[...] = idxf.astype(jnp.int32).reshape(1, 1, BLK)

    @pl.when(i == 0)
    def _():
        acc_ref[0] = 0.0

    acc_ref[0] += jnp.sum(m)

    @pl.when(i == pl.num_programs(0) - 1)
    def _():
        loss_ref[...] = jnp.full((1, 1), acc_ref[0], dtype=jnp.float32)


def _tc_argmin_loss(xt, weight, k):
    g = CHT // BLK
    return pl.pallas_call(
        _tc_body,
        grid=(g,),
        in_specs=[
            pl.BlockSpec((DIM, BLK), lambda i, k=k: (0, i + k * g)),
            pl.BlockSpec((N_EMB, DIM), lambda i: (0, 0)),
        ],
        out_specs=[
            pl.BlockSpec((1, 1, BLK), lambda i: (i, 0, 0)),
            pl.BlockSpec((1, 1), lambda i: (0, 0)),
        ],
        out_shape=[
            jax.ShapeDtypeStruct((g, 1, BLK), jnp.int32),
            jax.ShapeDtypeStruct((1, 1), jnp.float32),
        ],
        scratch_shapes=[pltpu.SMEM((1,), jnp.float32)],
    )(xt, weight)


def _sc_gather(w_pad, idx):
    n = idx.shape[0]
    rows_per_w = n // SC_NW
    n_ch = rows_per_w // SC_CH
    mesh = plsc.VectorSubcoreMesh(core_axis_name="c", subcore_axis_name="s")

    nb = 4

    @pl.kernel(out_type=jax.ShapeDtypeStruct((n, 128), jnp.float32),
               mesh=mesh,
               scratch_types=(
                   [pltpu.VMEM((rows_per_w,), jnp.int32)]
                   + [pltpu.VMEM((SC_CH, 128), jnp.float32)] * nb
                   + [pltpu.SemaphoreType.DMA] * (2 * nb)
               ))
    def k(w_hbm, i_hbm, o_hbm, idx_all, *bufs_sems):
        bufs = bufs_sems[:nb]
        gsems = bufs_sems[nb:2 * nb]
        wsems = bufs_sems[2 * nb:]
        wid = jax.lax.axis_index("s") * SC_NC + jax.lax.axis_index("c")
        base = wid * rows_per_w
        pltpu.sync_copy(i_hbm.at[pl.ds(base, rows_per_w)], idx_all)
        gathers = [None] * n_ch
        writes = [None] * n_ch
        for c in range(n_ch):
            s = c % nb
            if c >= nb:
                writes[c - nb].wait()
            gathers[c] = pltpu.async_copy(
                w_hbm.at[idx_all.at[pl.ds(c * SC_CH, SC_CH)]], bufs[s],
                gsems[s])
            if c >= 1:
                d = c - 1
                gathers[d].wait()
                writes[d] = pltpu.async_copy(
                    bufs[d % nb],
                    o_hbm.at[pl.ds(base + d * SC_CH, SC_CH)],
                    wsems[d % nb])
        d = n_ch - 1
        gathers[d].wait()
        writes[d] = pltpu.async_copy(
            bufs[d % nb], o_hbm.at[pl.ds(base + d * SC_CH, SC_CH)],
            wsems[d % nb])
        for d in range(max(0, n_ch - nb), n_ch):
            writes[d].wait()

    return k(w_pad, idx)


SL_BLK = 8192


def _packT_body(_, q_ref, o_ref):
    o_ref[...] = q_ref[:, :DIM].T


def _packT_first(q_raw):
    g = CHT // SL_BLK
    return pl.pallas_call(
        lambda q, o: _packT_body(None, q, o),
        grid=(g,),
        in_specs=[pl.BlockSpec((SL_BLK, 128), lambda i: (i, 0))],
        out_specs=pl.BlockSpec((DIM, SL_BLK), lambda i: (0, i)),
        out_shape=jax.ShapeDtypeStruct((DIM, N_TOK), jnp.float32),
    )(q_raw)


def _packT_into(buf, q_raw, k):
    g = CHT // SL_BLK
    return pl.pallas_call(
        _packT_body,
        grid=(g,),
        in_specs=[
            pl.BlockSpec(memory_space=pl.ANY),
            pl.BlockSpec((SL_BLK, 128), lambda i: (i, 0)),
        ],
        out_specs=pl.BlockSpec((DIM, SL_BLK), lambda i, k=k, g=g: (0, i + k * g)),
        out_shape=jax.ShapeDtypeStruct((DIM, N_TOK), jnp.float32),
        input_output_aliases={0: 0},
    )(buf, q_raw)


def kernel(inputs, weight):
    w_pad = jnp.concatenate(
        [weight, jnp.zeros((N_EMB, 128 - DIM), jnp.float32)], axis=1)
    xt = inputs.T
    idx_parts, loss_parts, q_raws = [], [], []
    for k in range(NCHUNK):
        idx2d, lsum = _tc_argmin_loss(xt, weight, k)
        idx = idx2d.reshape(CHT)
        q_raws.append(_sc_gather(w_pad, idx))
        idx_parts.append(idx)
        loss_parts.append(lsum[0, 0])
    qt = _packT_first(q_raws[0])
    for k in range(1, NCHUNK):
        qt = _packT_into(qt, q_raws[k], k)
    quantized = qt.T
    loss = sum(loss_parts) * (1.25 / (N_TOK * DIM))
    indices = jnp.concatenate(idx_parts, axis=0)
    return loss, quantized, indices
